# baseline (speedup 1.0000x reference)
import functools

import jax
import jax.numpy as jnp
from jax import lax
from jax.experimental import pallas as pl
from jax.experimental.pallas import tpu as pltpu
from jax.experimental.pallas import tpu_sc as plsc

_NUM_CLASSES = 19
_STRENGTH = 1.0
_LANES = 16


def _sc_gamma_body(labels_hbm, ranks_hbm, gamma_hbm, tab_v, lab_v, gam_v,
                   sem_t, sem_l, sem_o, *, per_tile, num_cores):
    wid = lax.axis_index("s") * num_cores + lax.axis_index("c")
    base = wid * per_tile

    copy_tab = pltpu.async_copy(ranks_hbm, tab_v, sem_t)
    copy_lab = pltpu.async_copy(labels_hbm.at[pl.ds(base, per_tile)], lab_v,
                                sem_l)

    copy_tab.wait()
    scale = _STRENGTH / (_NUM_CLASSES - 1)
    for j in range(2):
        r = tab_v[pl.ds(j * _LANES, _LANES)]
        tab_v[pl.ds(j * _LANES, _LANES)] = (1.0 + _STRENGTH) - r * scale

    copy_lab.wait()
    half = per_tile // 2
    out_copies = []
    for k in range(2):
        def body(i, _, k=k):
            off = k * half + i * _LANES
            idx = lab_v[pl.ds(off, _LANES)]
            gam_v[pl.ds(off, _LANES)] = plsc.load_gather(tab_v, [idx])
            return 0

        lax.fori_loop(0, half // _LANES, body, 0, unroll=8)
        out_copies.append(
            pltpu.async_copy(gam_v.at[pl.ds(k * half, half)],
                             gamma_hbm.at[pl.ds(base + k * half, half)],
                             sem_o))
    for c in out_copies:
        c.wait()


def _sc_gamma(labels_flat, ranks_padded):
    n = labels_flat.shape[0]
    info = plsc.get_sparse_core_info()
    nw = info.num_cores * info.num_subcores
    per_tile = n // nw
    mesh = plsc.VectorSubcoreMesh(core_axis_name="c", subcore_axis_name="s")
    k = functools.partial(
        pl.kernel,
        out_type=jax.ShapeDtypeStruct((n,), jnp.float32),
        mesh=mesh,
        scratch_types=[
            pltpu.VMEM((2 * _LANES,), jnp.float32),
            pltpu.VMEM((per_tile,), jnp.int32),
            pltpu.VMEM((per_tile,), jnp.float32),
            pltpu.SemaphoreType.DMA,
            pltpu.SemaphoreType.DMA,
            pltpu.SemaphoreType.DMA,
        ],
        compiler_params=pltpu.CompilerParams(needs_layout_passes=False),
    )(functools.partial(_sc_gamma_body, per_tile=per_tile,
                        num_cores=info.num_cores))
    return k(labels_flat, ranks_padded)


def _scale_body(g_ref, f_ref, o_ref):
    o_ref[...] = f_ref[...] * g_ref[...]


def _scale(features, gamma):
    b, c, h, w = features.shape
    s = 8
    m = h * w // s
    f2 = features.reshape(b * c, s, m)
    g2 = gamma.reshape(b, s, m)
    out = pl.pallas_call(
        _scale_body,
        grid=(b * c,),
        in_specs=[
            pl.BlockSpec((1, s, m), lambda i: (i // c, 0, 0)),
            pl.BlockSpec((1, s, m), lambda i: (i, 0, 0)),
        ],
        out_specs=pl.BlockSpec((1, s, m), lambda i: (i, 0, 0)),
        out_shape=jax.ShapeDtypeStruct((b * c, s, m), jnp.float32),
    )(g2, f2)
    return out.reshape(b, c, h, w)


def kernel(features, pseudo_labels, global_class_ranks):
    b, c, h, w = features.shape
    labels = pseudo_labels.reshape(-1).astype(jnp.int32)
    ranks = jnp.pad(global_class_ranks.astype(jnp.float32),
                    (0, 2 * _LANES - _NUM_CLASSES))
    gamma = _sc_gamma(labels, ranks).reshape(b, 1, h, w)
    return _scale(features, gamma)


# speedup vs baseline: 4.2644x; 4.2644x over previous
import functools

import jax
import jax.numpy as jnp
from jax import lax
from jax.experimental import pallas as pl
from jax.experimental.pallas import tpu as pltpu
from jax.experimental.pallas import tpu_sc as plsc

_NUM_CLASSES = 19
_STRENGTH = 1.0
_LANES = 16


def _sc_gamma_body(labels_hbm, ranks_hbm, gamma_hbm, tab_v, lab_v, gam_v,
                   sem_t, sem_l, sem_o, *, per_tile, num_cores):
    wid = lax.axis_index("s") * num_cores + lax.axis_index("c")
    base = wid * per_tile

    copy_tab = pltpu.async_copy(ranks_hbm, tab_v, sem_t)
    copy_lab = pltpu.async_copy(labels_hbm.at[pl.ds(base, per_tile)], lab_v,
                                sem_l)

    copy_tab.wait()
    scale = _STRENGTH / (_NUM_CLASSES - 1)
    for j in range(2):
        r = tab_v[pl.ds(j * _LANES, _LANES)]
        tab_v[pl.ds(j * _LANES, _LANES)] = (1.0 + _STRENGTH) - r * scale

    copy_lab.wait()
    half = per_tile // 2
    out_copies = []
    for k in range(2):
        def body(i, _, k=k):
            off = k * half + i * _LANES
            idx = lab_v[pl.ds(off, _LANES)]
            gam_v[pl.ds(off, _LANES)] = plsc.load_gather(tab_v, [idx])
            return 0

        lax.fori_loop(0, half // _LANES, body, 0, unroll=8)
        out_copies.append(
            pltpu.async_copy(gam_v.at[pl.ds(k * half, half)],
                             gamma_hbm.at[pl.ds(base + k * half, half)],
                             sem_o))
    for c in out_copies:
        c.wait()


def _sc_gamma(labels_flat, ranks_padded):
    n = labels_flat.shape[0]
    info = plsc.get_sparse_core_info()
    nw = info.num_cores * info.num_subcores
    per_tile = n // nw
    mesh = plsc.VectorSubcoreMesh(core_axis_name="c", subcore_axis_name="s")
    k = functools.partial(
        pl.kernel,
        out_type=jax.ShapeDtypeStruct((n,), jnp.float32),
        mesh=mesh,
        scratch_types=[
            pltpu.VMEM((2 * _LANES,), jnp.float32),
            pltpu.VMEM((per_tile,), jnp.int32),
            pltpu.VMEM((per_tile,), jnp.float32),
            pltpu.SemaphoreType.DMA,
            pltpu.SemaphoreType.DMA,
            pltpu.SemaphoreType.DMA,
        ],
        compiler_params=pltpu.CompilerParams(needs_layout_passes=False),
    )(functools.partial(_sc_gamma_body, per_tile=per_tile,
                        num_cores=info.num_cores))
    return k(labels_flat, ranks_padded)


def _scale_body(g_ref, f_ref, o_ref):
    o_ref[...] = f_ref[...] * g_ref[...]


def _scale(features, gamma):
    b, c, h, w = features.shape
    hb = 64
    return pl.pallas_call(
        _scale_body,
        grid=(b, h // hb),
        in_specs=[
            pl.BlockSpec((1, 1, hb, w), lambda i, j: (i, 0, j, 0)),
            pl.BlockSpec((1, c, hb, w), lambda i, j: (i, 0, j, 0)),
        ],
        out_specs=pl.BlockSpec((1, c, hb, w), lambda i, j: (i, 0, j, 0)),
        out_shape=jax.ShapeDtypeStruct((b, c, h, w), jnp.float32),
    )(gamma, features)


def kernel(features, pseudo_labels, global_class_ranks):
    b, c, h, w = features.shape
    labels = pseudo_labels.reshape(-1).astype(jnp.int32)
    ranks = jnp.pad(global_class_ranks.astype(jnp.float32),
                    (0, 2 * _LANES - _NUM_CLASSES))
    gamma = _sc_gamma(labels, ranks).reshape(b, 1, h, w)
    return _scale(features, gamma)


# speedup vs baseline: 4.2691x; 1.0011x over previous
import functools

import jax
import jax.numpy as jnp
from jax import lax
from jax.experimental import pallas as pl
from jax.experimental.pallas import tpu as pltpu
from jax.experimental.pallas import tpu_sc as plsc

_NUM_CLASSES = 19
_STRENGTH = 1.0
_LANES = 16


def _sc_gamma_body(labels_hbm, ranks_hbm, gamma_hbm, tab_v, lab_v, gam_v,
                   sem_t, sem_l, sem_o, *, per_tile, num_cores):
    wid = lax.axis_index("s") * num_cores + lax.axis_index("c")
    base = wid * per_tile

    nchunks = 4
    chunk = per_tile // nchunks
    copy_tab = pltpu.async_copy(ranks_hbm, tab_v, sem_t)
    lab_copies = [
        pltpu.async_copy(labels_hbm.at[pl.ds(base + k * chunk, chunk)],
                         lab_v.at[pl.ds(k * chunk, chunk)], sem_l)
        for k in range(nchunks)
    ]

    copy_tab.wait()
    scale = _STRENGTH / (_NUM_CLASSES - 1)
    for j in range(2):
        r = tab_v[pl.ds(j * _LANES, _LANES)]
        tab_v[pl.ds(j * _LANES, _LANES)] = (1.0 + _STRENGTH) - r * scale

    out_copies = []
    for k in range(nchunks):
        lab_copies[k].wait()

        def body(i, _, k=k):
            off = k * chunk + i * _LANES
            idx = lab_v[pl.ds(off, _LANES)]
            gam_v[pl.ds(off, _LANES)] = plsc.load_gather(tab_v, [idx])
            return 0

        lax.fori_loop(0, chunk // _LANES, body, 0, unroll=8)
        out_copies.append(
            pltpu.async_copy(gam_v.at[pl.ds(k * chunk, chunk)],
                             gamma_hbm.at[pl.ds(base + k * chunk, chunk)],
                             sem_o))
    for c in out_copies:
        c.wait()


def _sc_gamma(labels_flat, ranks_padded):
    n = labels_flat.shape[0]
    info = plsc.get_sparse_core_info()
    nw = info.num_cores * info.num_subcores
    per_tile = n // nw
    mesh = plsc.VectorSubcoreMesh(core_axis_name="c", subcore_axis_name="s")
    k = functools.partial(
        pl.kernel,
        out_type=jax.ShapeDtypeStruct((n,), jnp.float32),
        mesh=mesh,
        scratch_types=[
            pltpu.VMEM((2 * _LANES,), jnp.float32),
            pltpu.VMEM((per_tile,), jnp.int32),
            pltpu.VMEM((per_tile,), jnp.float32),
            pltpu.SemaphoreType.DMA,
            pltpu.SemaphoreType.DMA,
            pltpu.SemaphoreType.DMA,
        ],
        compiler_params=pltpu.CompilerParams(needs_layout_passes=False),
    )(functools.partial(_sc_gamma_body, per_tile=per_tile,
                        num_cores=info.num_cores))
    return k(labels_flat, ranks_padded)


def _scale_body(g_ref, f_ref, o_ref):
    o_ref[...] = f_ref[...] * g_ref[...]


def _scale(features, gamma):
    b, c, h, w = features.shape
    hb = 64
    return pl.pallas_call(
        _scale_body,
        grid=(b, h // hb),
        in_specs=[
            pl.BlockSpec((1, 1, hb, w), lambda i, j: (i, 0, j, 0)),
            pl.BlockSpec((1, c, hb, w), lambda i, j: (i, 0, j, 0)),
        ],
        out_specs=pl.BlockSpec((1, c, hb, w), lambda i, j: (i, 0, j, 0)),
        out_shape=jax.ShapeDtypeStruct((b, c, h, w), jnp.float32),
    )(gamma, features)


def kernel(features, pseudo_labels, global_class_ranks):
    b, c, h, w = features.shape
    labels = pseudo_labels.reshape(-1).astype(jnp.int32)
    ranks = jnp.pad(global_class_ranks.astype(jnp.float32),
                    (0, 2 * _LANES - _NUM_CLASSES))
    gamma = _sc_gamma(labels, ranks).reshape(b, 1, h, w)
    return _scale(features, gamma)
